# Initial kernel scaffold; baseline (speedup 1.0000x reference)
import jax
import jax.numpy as jnp
from jax.experimental import pallas as pl


def kernel(images, gt_boxes, gt_classes, anchor_boxes):
    raise NotImplementedError("write your pallas kernel here")


# speedup vs baseline: 2.8073x; 2.8073x over previous
import functools

import jax
import jax.numpy as jnp
from jax.experimental import pallas as pl


_T = 512
_L = 128


def _encode_kernel(n_real_gt, a_ref, gtc_ref, gtp_ref, box_ref, cls_ref):
    a = a_ref[...]
    gtc = gtc_ref[0]
    gtp = gtp_ref[0]

    ax1 = a[:, 0:1]
    ay1 = a[:, 1:2]
    ax2 = a[:, 2:3]
    ay2 = a[:, 3:4]

    gx1 = gtc[0:1, :]
    gy1 = gtc[1:2, :]
    gx2 = gtc[2:3, :]
    gy2 = gtc[3:4, :]

    ltx = jnp.maximum(ax1, gx1)
    lty = jnp.maximum(ay1, gy1)
    rbx = jnp.minimum(ax2, gx2)
    rby = jnp.minimum(ay2, gy2)
    wx = jnp.maximum(rbx - ltx, 0.0)
    wy = jnp.maximum(rby - lty, 0.0)
    inter = wx * wy
    area_a = (ax2 - ax1) * (ay2 - ay1)
    area_b = (gx2 - gx1) * (gy2 - gy1)
    union = area_a + area_b - inter
    iou = inter / jnp.maximum(union, 1e-8)

    lane = jax.lax.broadcasted_iota(jnp.int32, (_T, _L), 1)
    iou = jnp.where(lane < n_real_gt, iou, -1.0)

    max_iou = jnp.max(iou, axis=1, keepdims=True)
    idx = jnp.min(jnp.where(iou == max_iou, lane, _L), axis=1, keepdims=True)
    onehot = (lane == idx).astype(jnp.float32)

    matched = jnp.dot(
        onehot,
        gtp,
        preferred_element_type=jnp.float32,
        precision=jax.lax.Precision.HIGHEST,
    )
    mx1 = matched[:, 0:1]
    my1 = matched[:, 1:2]
    mx2 = matched[:, 2:3]
    my2 = matched[:, 3:4]
    mcls = matched[:, 4:5]

    awx = ax2 - ax1
    awy = ay2 - ay1
    acx = ax1 + awx * 0.5
    acy = ay1 + awy * 0.5
    gwx = mx2 - mx1
    gwy = my2 - my1
    gcx = mx1 + gwx * 0.5
    gcy = my1 + gwy * 0.5

    tx = ((gcx - acx) / awx) / 0.1
    ty = ((gcy - acy) / awy) / 0.1
    tw = jnp.log(gwx / awx) / 0.2
    th = jnp.log(gwy / awy) / 0.2
    box = jnp.concatenate([tx, ty, tw, th], axis=1)
    box = jnp.where(jnp.isnan(box), -2.0, box)

    positive = max_iou >= 0.5
    ignore = jnp.logical_and(max_iou >= 0.4, max_iou < 0.5)
    cls = jnp.where(positive, mcls, -1.0)
    cls = jnp.where(ignore, -2.0, cls)
    cls = jnp.where(jnp.isnan(cls), -2.0, cls)

    box_ref[0] = box
    cls_ref[0] = cls


@jax.jit
def kernel(images, gt_boxes, gt_classes, anchor_boxes):
    del images
    B, N, _ = gt_boxes.shape
    A = anchor_boxes.shape[0]
    Ap = ((A + _T - 1) // _T) * _T
    G = Ap // _T

    anchors_pad = jnp.pad(anchor_boxes, ((0, Ap - A), (0, 0)))

    gt_pad = jnp.pad(gt_boxes, ((0, 0), (0, _L - N), (0, 0)))
    cls_pad = jnp.pad(gt_classes, ((0, 0), (0, _L - N)))
    gtc = jnp.concatenate(
        [
            jnp.transpose(gt_pad, (0, 2, 1)),
            jnp.zeros((B, 4, _L), jnp.float32),
        ],
        axis=1,
    )
    gtp = jnp.concatenate(
        [gt_pad, cls_pad[..., None], jnp.zeros((B, _L, 3), jnp.float32)],
        axis=-1,
    )

    box_out, cls_out = pl.pallas_call(
        functools.partial(_encode_kernel, N),
        grid=(B, G),
        in_specs=[
            pl.BlockSpec((_T, 4), lambda b, g: (g, 0)),
            pl.BlockSpec((1, 8, _L), lambda b, g: (b, 0, 0)),
            pl.BlockSpec((1, _L, 8), lambda b, g: (b, 0, 0)),
        ],
        out_specs=[
            pl.BlockSpec((1, _T, 4), lambda b, g: (b, g, 0)),
            pl.BlockSpec((1, _T, 1), lambda b, g: (b, g, 0)),
        ],
        out_shape=[
            jax.ShapeDtypeStruct((B, Ap, 4), jnp.float32),
            jax.ShapeDtypeStruct((B, Ap, 1), jnp.float32),
        ],
    )(anchors_pad, gtc, gtp)

    return box_out[:, :A, :], cls_out[:, :A, 0]


# speedup vs baseline: 3.2785x; 1.1678x over previous
import functools

import jax
import jax.numpy as jnp
from jax.experimental import pallas as pl


_T = 1024
_L = 128


def _encode_kernel(n_real_gt, a_ref, gtc_ref, gtp_ref, box_ref, cls_ref):
    a = a_ref[...]
    gtc = gtc_ref[0]
    gtp = gtp_ref[0]

    ax1 = a[:, 0:1]
    ay1 = a[:, 1:2]
    ax2 = a[:, 2:3]
    ay2 = a[:, 3:4]

    gx1 = gtc[0:1, :]
    gy1 = gtc[1:2, :]
    gx2 = gtc[2:3, :]
    gy2 = gtc[3:4, :]

    ltx = jnp.maximum(ax1, gx1)
    lty = jnp.maximum(ay1, gy1)
    rbx = jnp.minimum(ax2, gx2)
    rby = jnp.minimum(ay2, gy2)
    wx = jnp.maximum(rbx - ltx, 0.0)
    wy = jnp.maximum(rby - lty, 0.0)
    inter = wx * wy
    area_a = (ax2 - ax1) * (ay2 - ay1)
    area_b = (gx2 - gx1) * (gy2 - gy1)
    union = area_a + area_b - inter
    iou = inter / jnp.maximum(union, 1e-8)

    lane = jax.lax.broadcasted_iota(jnp.int32, (_T, _L), 1)
    iou = jnp.where(lane < n_real_gt, iou, -1.0)

    max_iou = jnp.max(iou, axis=1, keepdims=True)
    idx = jnp.min(jnp.where(iou == max_iou, lane, _L), axis=1, keepdims=True)
    onehot = (lane == idx).astype(jnp.float32)

    matched = jnp.dot(
        onehot,
        gtp,
        preferred_element_type=jnp.float32,
        precision=jax.lax.Precision.HIGHEST,
    )
    mx1 = matched[:, 0:1]
    my1 = matched[:, 1:2]
    mx2 = matched[:, 2:3]
    my2 = matched[:, 3:4]
    mcls = matched[:, 4:5]

    awx = ax2 - ax1
    awy = ay2 - ay1
    acx = ax1 + awx * 0.5
    acy = ay1 + awy * 0.5
    gwx = mx2 - mx1
    gwy = my2 - my1
    gcx = mx1 + gwx * 0.5
    gcy = my1 + gwy * 0.5

    tx = ((gcx - acx) / awx) / 0.1
    ty = ((gcy - acy) / awy) / 0.1
    tw = jnp.log(gwx / awx) / 0.2
    th = jnp.log(gwy / awy) / 0.2
    box = jnp.concatenate([tx, ty, tw, th], axis=1)
    box = jnp.where(jnp.isnan(box), -2.0, box)

    positive = max_iou >= 0.5
    ignore = jnp.logical_and(max_iou >= 0.4, max_iou < 0.5)
    cls = jnp.where(positive, mcls, -1.0)
    cls = jnp.where(ignore, -2.0, cls)
    cls = jnp.where(jnp.isnan(cls), -2.0, cls)

    box_ref[0] = box
    cls_ref[0] = cls


@jax.jit
def kernel(images, gt_boxes, gt_classes, anchor_boxes):
    del images
    B, N, _ = gt_boxes.shape
    A = anchor_boxes.shape[0]
    Ap = ((A + _T - 1) // _T) * _T
    G = Ap // _T

    anchors_pad = jnp.pad(anchor_boxes, ((0, Ap - A), (0, 0)))

    gt_pad = jnp.pad(gt_boxes, ((0, 0), (0, _L - N), (0, 0)))
    cls_pad = jnp.pad(gt_classes, ((0, 0), (0, _L - N)))
    gtc = jnp.concatenate(
        [
            jnp.transpose(gt_pad, (0, 2, 1)),
            jnp.zeros((B, 4, _L), jnp.float32),
        ],
        axis=1,
    )
    gtp = jnp.concatenate(
        [gt_pad, cls_pad[..., None], jnp.zeros((B, _L, 3), jnp.float32)],
        axis=-1,
    )

    box_out, cls_out = pl.pallas_call(
        functools.partial(_encode_kernel, N),
        grid=(B, G),
        in_specs=[
            pl.BlockSpec((_T, 4), lambda b, g: (g, 0)),
            pl.BlockSpec((1, 8, _L), lambda b, g: (b, 0, 0)),
            pl.BlockSpec((1, _L, 8), lambda b, g: (b, 0, 0)),
        ],
        out_specs=[
            pl.BlockSpec((1, _T, 4), lambda b, g: (b, g, 0)),
            pl.BlockSpec((1, _T, 1), lambda b, g: (b, g, 0)),
        ],
        out_shape=[
            jax.ShapeDtypeStruct((B, Ap, 4), jnp.float32),
            jax.ShapeDtypeStruct((B, Ap, 1), jnp.float32),
        ],
    )(anchors_pad, gtc, gtp)

    return box_out[:, :A, :], cls_out[:, :A, 0]


# speedup vs baseline: 3.4873x; 1.0637x over previous
import functools

import jax
import jax.numpy as jnp
from jax.experimental import pallas as pl


_T = 2048
_L = 128


def _encode_kernel(n_real_gt, a_ref, gtc_ref, gtp_ref, box_ref, cls_ref):
    a = a_ref[...]
    gtc = gtc_ref[0]
    gtp = gtp_ref[0]

    ax1 = a[:, 0:1]
    ay1 = a[:, 1:2]
    ax2 = a[:, 2:3]
    ay2 = a[:, 3:4]

    gx1 = gtc[0:1, :]
    gy1 = gtc[1:2, :]
    gx2 = gtc[2:3, :]
    gy2 = gtc[3:4, :]

    ltx = jnp.maximum(ax1, gx1)
    lty = jnp.maximum(ay1, gy1)
    rbx = jnp.minimum(ax2, gx2)
    rby = jnp.minimum(ay2, gy2)
    wx = jnp.maximum(rbx - ltx, 0.0)
    wy = jnp.maximum(rby - lty, 0.0)
    inter = wx * wy
    area_a = (ax2 - ax1) * (ay2 - ay1)
    area_b = (gx2 - gx1) * (gy2 - gy1)
    union = area_a + area_b - inter
    iou = inter / jnp.maximum(union, 1e-8)

    lane = jax.lax.broadcasted_iota(jnp.int32, (_T, _L), 1)
    iou = jnp.where(lane < n_real_gt, iou, -1.0)

    max_iou = jnp.max(iou, axis=1, keepdims=True)
    idx = jnp.min(jnp.where(iou == max_iou, lane, _L), axis=1, keepdims=True)
    onehot = (lane == idx).astype(jnp.float32)

    matched = jnp.dot(
        onehot,
        gtp,
        preferred_element_type=jnp.float32,
        precision=jax.lax.Precision.HIGHEST,
    )
    mx1 = matched[:, 0:1]
    my1 = matched[:, 1:2]
    mx2 = matched[:, 2:3]
    my2 = matched[:, 3:4]
    mcls = matched[:, 4:5]

    awx = ax2 - ax1
    awy = ay2 - ay1
    acx = ax1 + awx * 0.5
    acy = ay1 + awy * 0.5
    gwx = mx2 - mx1
    gwy = my2 - my1
    gcx = mx1 + gwx * 0.5
    gcy = my1 + gwy * 0.5

    tx = ((gcx - acx) / awx) / 0.1
    ty = ((gcy - acy) / awy) / 0.1
    tw = jnp.log(gwx / awx) / 0.2
    th = jnp.log(gwy / awy) / 0.2
    box = jnp.concatenate([tx, ty, tw, th], axis=1)
    box = jnp.where(jnp.isnan(box), -2.0, box)

    positive = max_iou >= 0.5
    ignore = jnp.logical_and(max_iou >= 0.4, max_iou < 0.5)
    cls = jnp.where(positive, mcls, -1.0)
    cls = jnp.where(ignore, -2.0, cls)
    cls = jnp.where(jnp.isnan(cls), -2.0, cls)

    box_ref[0] = box
    cls_ref[0] = cls


@jax.jit
def kernel(images, gt_boxes, gt_classes, anchor_boxes):
    del images
    B, N, _ = gt_boxes.shape
    A = anchor_boxes.shape[0]
    Ap = ((A + _T - 1) // _T) * _T
    G = Ap // _T

    anchors_pad = jnp.pad(anchor_boxes, ((0, Ap - A), (0, 0)))

    gt_pad = jnp.pad(gt_boxes, ((0, 0), (0, _L - N), (0, 0)))
    cls_pad = jnp.pad(gt_classes, ((0, 0), (0, _L - N)))
    gtc = jnp.concatenate(
        [
            jnp.transpose(gt_pad, (0, 2, 1)),
            jnp.zeros((B, 4, _L), jnp.float32),
        ],
        axis=1,
    )
    gtp = jnp.concatenate(
        [gt_pad, cls_pad[..., None], jnp.zeros((B, _L, 3), jnp.float32)],
        axis=-1,
    )

    box_out, cls_out = pl.pallas_call(
        functools.partial(_encode_kernel, N),
        grid=(B, G),
        in_specs=[
            pl.BlockSpec((_T, 4), lambda b, g: (g, 0)),
            pl.BlockSpec((1, 8, _L), lambda b, g: (b, 0, 0)),
            pl.BlockSpec((1, _L, 8), lambda b, g: (b, 0, 0)),
        ],
        out_specs=[
            pl.BlockSpec((1, _T, 4), lambda b, g: (b, g, 0)),
            pl.BlockSpec((1, _T, 1), lambda b, g: (b, g, 0)),
        ],
        out_shape=[
            jax.ShapeDtypeStruct((B, Ap, 4), jnp.float32),
            jax.ShapeDtypeStruct((B, Ap, 1), jnp.float32),
        ],
    )(anchors_pad, gtc, gtp)

    return box_out[:, :A, :], cls_out[:, :A, 0]


# speedup vs baseline: 3.6000x; 1.0323x over previous
import functools

import jax
import jax.numpy as jnp
from jax.experimental import pallas as pl


_T = 4096
_L = 128


def _encode_kernel(n_real_gt, a_ref, gtc_ref, gtp_ref, box_ref, cls_ref):
    a = a_ref[...]
    gtc = gtc_ref[0]
    gtp = gtp_ref[0]

    ax1 = a[:, 0:1]
    ay1 = a[:, 1:2]
    ax2 = a[:, 2:3]
    ay2 = a[:, 3:4]

    gx1 = gtc[0:1, :]
    gy1 = gtc[1:2, :]
    gx2 = gtc[2:3, :]
    gy2 = gtc[3:4, :]

    ltx = jnp.maximum(ax1, gx1)
    lty = jnp.maximum(ay1, gy1)
    rbx = jnp.minimum(ax2, gx2)
    rby = jnp.minimum(ay2, gy2)
    wx = jnp.maximum(rbx - ltx, 0.0)
    wy = jnp.maximum(rby - lty, 0.0)
    inter = wx * wy
    area_a = (ax2 - ax1) * (ay2 - ay1)
    area_b = (gx2 - gx1) * (gy2 - gy1)
    union = area_a + area_b - inter
    iou = inter / jnp.maximum(union, 1e-8)

    lane = jax.lax.broadcasted_iota(jnp.int32, (_T, _L), 1)
    iou = jnp.where(lane < n_real_gt, iou, -1.0)

    max_iou = jnp.max(iou, axis=1, keepdims=True)
    idx = jnp.min(jnp.where(iou == max_iou, lane, _L), axis=1, keepdims=True)
    onehot = (lane == idx).astype(jnp.float32)

    matched = jnp.dot(
        onehot,
        gtp,
        preferred_element_type=jnp.float32,
        precision=jax.lax.Precision.HIGHEST,
    )
    mx1 = matched[:, 0:1]
    my1 = matched[:, 1:2]
    mx2 = matched[:, 2:3]
    my2 = matched[:, 3:4]
    mcls = matched[:, 4:5]

    awx = ax2 - ax1
    awy = ay2 - ay1
    acx = ax1 + awx * 0.5
    acy = ay1 + awy * 0.5
    gwx = mx2 - mx1
    gwy = my2 - my1
    gcx = mx1 + gwx * 0.5
    gcy = my1 + gwy * 0.5

    tx = ((gcx - acx) / awx) / 0.1
    ty = ((gcy - acy) / awy) / 0.1
    tw = jnp.log(gwx / awx) / 0.2
    th = jnp.log(gwy / awy) / 0.2
    box = jnp.concatenate([tx, ty, tw, th], axis=1)
    box = jnp.where(jnp.isnan(box), -2.0, box)

    positive = max_iou >= 0.5
    ignore = jnp.logical_and(max_iou >= 0.4, max_iou < 0.5)
    cls = jnp.where(positive, mcls, -1.0)
    cls = jnp.where(ignore, -2.0, cls)
    cls = jnp.where(jnp.isnan(cls), -2.0, cls)

    box_ref[0] = box
    cls_ref[0] = cls


@jax.jit
def kernel(images, gt_boxes, gt_classes, anchor_boxes):
    del images
    B, N, _ = gt_boxes.shape
    A = anchor_boxes.shape[0]
    Ap = ((A + _T - 1) // _T) * _T
    G = Ap // _T

    anchors_pad = jnp.pad(anchor_boxes, ((0, Ap - A), (0, 0)))

    gt_pad = jnp.pad(gt_boxes, ((0, 0), (0, _L - N), (0, 0)))
    cls_pad = jnp.pad(gt_classes, ((0, 0), (0, _L - N)))
    gtc = jnp.concatenate(
        [
            jnp.transpose(gt_pad, (0, 2, 1)),
            jnp.zeros((B, 4, _L), jnp.float32),
        ],
        axis=1,
    )
    gtp = jnp.concatenate(
        [gt_pad, cls_pad[..., None], jnp.zeros((B, _L, 3), jnp.float32)],
        axis=-1,
    )

    box_out, cls_out = pl.pallas_call(
        functools.partial(_encode_kernel, N),
        grid=(B, G),
        in_specs=[
            pl.BlockSpec((_T, 4), lambda b, g: (g, 0)),
            pl.BlockSpec((1, 8, _L), lambda b, g: (b, 0, 0)),
            pl.BlockSpec((1, _L, 8), lambda b, g: (b, 0, 0)),
        ],
        out_specs=[
            pl.BlockSpec((1, _T, 4), lambda b, g: (b, g, 0)),
            pl.BlockSpec((1, _T, 1), lambda b, g: (b, g, 0)),
        ],
        out_shape=[
            jax.ShapeDtypeStruct((B, Ap, 4), jnp.float32),
            jax.ShapeDtypeStruct((B, Ap, 1), jnp.float32),
        ],
    )(anchors_pad, gtc, gtp)

    return box_out[:, :A, :], cls_out[:, :A, 0]


# speedup vs baseline: 3.6305x; 1.0085x over previous
import functools

import jax
import jax.numpy as jnp
from jax.experimental import pallas as pl


_T = 8192
_L = 128


def _encode_kernel(n_real_gt, a_ref, gtc_ref, gtp_ref, box_ref, cls_ref):
    a = a_ref[...]
    gtc = gtc_ref[0]
    gtp = gtp_ref[0]

    ax1 = a[:, 0:1]
    ay1 = a[:, 1:2]
    ax2 = a[:, 2:3]
    ay2 = a[:, 3:4]

    gx1 = gtc[0:1, :]
    gy1 = gtc[1:2, :]
    gx2 = gtc[2:3, :]
    gy2 = gtc[3:4, :]

    ltx = jnp.maximum(ax1, gx1)
    lty = jnp.maximum(ay1, gy1)
    rbx = jnp.minimum(ax2, gx2)
    rby = jnp.minimum(ay2, gy2)
    wx = jnp.maximum(rbx - ltx, 0.0)
    wy = jnp.maximum(rby - lty, 0.0)
    inter = wx * wy
    area_a = (ax2 - ax1) * (ay2 - ay1)
    area_b = (gx2 - gx1) * (gy2 - gy1)
    union = area_a + area_b - inter
    iou = inter / jnp.maximum(union, 1e-8)

    lane = jax.lax.broadcasted_iota(jnp.int32, (_T, _L), 1)
    iou = jnp.where(lane < n_real_gt, iou, -1.0)

    max_iou = jnp.max(iou, axis=1, keepdims=True)
    idx = jnp.min(jnp.where(iou == max_iou, lane, _L), axis=1, keepdims=True)
    onehot = (lane == idx).astype(jnp.float32)

    matched = jnp.dot(
        onehot,
        gtp,
        preferred_element_type=jnp.float32,
        precision=jax.lax.Precision.HIGHEST,
    )
    mx1 = matched[:, 0:1]
    my1 = matched[:, 1:2]
    mx2 = matched[:, 2:3]
    my2 = matched[:, 3:4]
    mcls = matched[:, 4:5]

    awx = ax2 - ax1
    awy = ay2 - ay1
    acx = ax1 + awx * 0.5
    acy = ay1 + awy * 0.5
    gwx = mx2 - mx1
    gwy = my2 - my1
    gcx = mx1 + gwx * 0.5
    gcy = my1 + gwy * 0.5

    tx = ((gcx - acx) / awx) / 0.1
    ty = ((gcy - acy) / awy) / 0.1
    tw = jnp.log(gwx / awx) / 0.2
    th = jnp.log(gwy / awy) / 0.2
    box = jnp.concatenate([tx, ty, tw, th], axis=1)
    box = jnp.where(jnp.isnan(box), -2.0, box)

    positive = max_iou >= 0.5
    ignore = jnp.logical_and(max_iou >= 0.4, max_iou < 0.5)
    cls = jnp.where(positive, mcls, -1.0)
    cls = jnp.where(ignore, -2.0, cls)
    cls = jnp.where(jnp.isnan(cls), -2.0, cls)

    box_ref[0] = box
    cls_ref[0] = cls


@jax.jit
def kernel(images, gt_boxes, gt_classes, anchor_boxes):
    del images
    B, N, _ = gt_boxes.shape
    A = anchor_boxes.shape[0]
    Ap = ((A + _T - 1) // _T) * _T
    G = Ap // _T

    anchors_pad = jnp.pad(anchor_boxes, ((0, Ap - A), (0, 0)))

    gt_pad = jnp.pad(gt_boxes, ((0, 0), (0, _L - N), (0, 0)))
    cls_pad = jnp.pad(gt_classes, ((0, 0), (0, _L - N)))
    gtc = jnp.concatenate(
        [
            jnp.transpose(gt_pad, (0, 2, 1)),
            jnp.zeros((B, 4, _L), jnp.float32),
        ],
        axis=1,
    )
    gtp = jnp.concatenate(
        [gt_pad, cls_pad[..., None], jnp.zeros((B, _L, 3), jnp.float32)],
        axis=-1,
    )

    box_out, cls_out = pl.pallas_call(
        functools.partial(_encode_kernel, N),
        grid=(B, G),
        in_specs=[
            pl.BlockSpec((_T, 4), lambda b, g: (g, 0)),
            pl.BlockSpec((1, 8, _L), lambda b, g: (b, 0, 0)),
            pl.BlockSpec((1, _L, 8), lambda b, g: (b, 0, 0)),
        ],
        out_specs=[
            pl.BlockSpec((1, _T, 4), lambda b, g: (b, g, 0)),
            pl.BlockSpec((1, _T, 1), lambda b, g: (b, g, 0)),
        ],
        out_shape=[
            jax.ShapeDtypeStruct((B, Ap, 4), jnp.float32),
            jax.ShapeDtypeStruct((B, Ap, 1), jnp.float32),
        ],
    )(anchors_pad, gtc, gtp)

    return box_out[:, :A, :], cls_out[:, :A, 0]
